# baseline (speedup 1.0000x reference)
import functools

import jax
import jax.numpy as jnp
from jax import lax
from jax.experimental import pallas as pl
from jax.experimental.pallas import tpu as pltpu
from jax.experimental.pallas import tpu_sc as plsc

_LANES = 16


def _make_emb_kernel(n_tok, D, T, tok_per_w, ch):
    n_chunks = tok_per_w // ch
    mesh = plsc.VectorSubcoreMesh(core_axis_name="c", subcore_axis_name="s")
    n_cores = mesh.num_cores

    @functools.partial(
        pl.kernel,
        mesh=mesh,
        out_type=jax.ShapeDtypeStruct((n_tok, D), jnp.float32),
        scratch_types=[
            pltpu.VMEM((tok_per_w,), jnp.int32),
            pltpu.VMEM((ch, D), jnp.float32),
            pltpu.VMEM((ch, D), jnp.float32),
            pltpu.SemaphoreType.DMA,
            pltpu.SemaphoreType.DMA,
        ],
    )
    def emb_kernel(idx_hbm, tok_hbm, pos_hbm, out_hbm, idx_v, rows_v, pos_v,
                   g_sem, p_sem):
        wid = lax.axis_index("s") * n_cores + lax.axis_index("c")
        base = wid * tok_per_w
        t0 = base % T
        pltpu.sync_copy(idx_hbm.at[pl.ds(base, tok_per_w)], idx_v)
        for c in range(n_chunks):
            g = pltpu.async_copy(
                tok_hbm.at[idx_v.at[pl.ds(c * ch, ch)]], rows_v, g_sem)
            p = pltpu.async_copy(
                pos_hbm.at[pl.ds(t0 + c * ch, ch)], pos_v, p_sem)
            g.wait()
            p.wait()

            def add_body(j, carry, _rows=rows_v, _pos=pos_v):
                col = j * _LANES
                for i in range(ch):
                    plsc.addupdate(_rows.at[i, pl.ds(col, _LANES)],
                                   _pos[i, pl.ds(col, _LANES)])
                return carry

            lax.fori_loop(0, D // _LANES, add_body, 0)
            pltpu.sync_copy(rows_v, out_hbm.at[pl.ds(base + c * ch, ch)])

    return emb_kernel


@jax.jit
def kernel(idx, tok_emb, pos_emb):
    b, t = idx.shape
    _, d = tok_emb.shape
    n_tok = b * t
    idx_flat = idx.reshape(n_tok).astype(jnp.int32)
    pos2d = pos_emb[0, :t, :].reshape(t, d)
    emb = _make_emb_kernel(n_tok, d, t, n_tok // 32, 16)
    out = emb(idx_flat, tok_emb, pos2d)
    return out.reshape(b, t, d)


# speedup vs baseline: 1.2929x; 1.2929x over previous
import functools

import jax
import jax.numpy as jnp
from jax import lax
from jax.experimental import pallas as pl
from jax.experimental.pallas import tpu as pltpu
from jax.experimental.pallas import tpu_sc as plsc

_LANES = 16


def _make_emb_kernel(B, T, D, n_workers):
    rw = T // n_workers
    mesh = plsc.VectorSubcoreMesh(core_axis_name="c", subcore_axis_name="s")
    n_cores = mesh.num_cores

    @functools.partial(
        pl.kernel,
        mesh=mesh,
        out_type=jax.ShapeDtypeStruct((B * T, D), jnp.float32),
        scratch_types=[
            pltpu.VMEM((B * rw,), jnp.int32),
            pltpu.VMEM((rw, D), jnp.float32),
            pltpu.VMEM((rw, D), jnp.float32),
            pltpu.VMEM((rw, D), jnp.float32),
            pltpu.SemaphoreType.DMA,
            pltpu.SemaphoreType.DMA,
            pltpu.SemaphoreType.DMA,
            pltpu.SemaphoreType.DMA,
            pltpu.SemaphoreType.DMA,
        ],
    )
    def emb_kernel(idx_hbm, tok_hbm, pos_hbm, out_hbm, idx_v, rows0, rows1,
                   pos_v, g_sem0, g_sem1, o_sem0, o_sem1, p_sem):
        wid = lax.axis_index("s") * n_cores + lax.axis_index("c")
        t0 = wid * rw
        bufs = (rows0, rows1)
        g_sems = (g_sem0, g_sem1)
        o_sems = (o_sem0, o_sem1)

        for b in range(B):
            pltpu.sync_copy(idx_hbm.at[pl.ds(b * T + t0, rw)],
                            idx_v.at[pl.ds(b * rw, rw)])
        p = pltpu.async_copy(pos_hbm.at[pl.ds(t0, rw)], pos_v, p_sem)

        def start_gather(b):
            return pltpu.async_copy(
                tok_hbm.at[idx_v.at[pl.ds(b * rw, rw)]], bufs[b % 2],
                g_sems[b % 2])

        gathers = [start_gather(0)]
        outs = [None, None]
        p.wait()
        for b in range(B):
            if b >= 2:
                outs[b % 2].wait()
            if b + 1 < B:
                gathers.append(start_gather(b + 1))
            gathers[b].wait()
            buf = bufs[b % 2]

            def add_body(j, carry, _buf=buf):
                col = j * _LANES
                for i in range(rw):
                    plsc.addupdate(_buf.at[i, pl.ds(col, _LANES)],
                                   pos_v[i, pl.ds(col, _LANES)])
                return carry

            lax.fori_loop(0, D // _LANES, add_body, 0)
            outs[b % 2] = pltpu.async_copy(
                buf, out_hbm.at[pl.ds(b * T + t0, rw)], o_sems[b % 2])
        outs[(B - 2) % 2].wait()
        outs[(B - 1) % 2].wait()

    return emb_kernel


@jax.jit
def kernel(idx, tok_emb, pos_emb):
    b, t = idx.shape
    _, d = tok_emb.shape
    idx_flat = idx.reshape(b * t).astype(jnp.int32)
    pos2d = pos_emb[0, :t, :].reshape(t, d)
    emb = _make_emb_kernel(b, t, d, 32)
    out = emb(idx_flat, tok_emb, pos2d)
    return out.reshape(b, t, d)


# speedup vs baseline: 1.3558x; 1.0487x over previous
import functools

import jax
import jax.numpy as jnp
from jax import lax
from jax.experimental import pallas as pl
from jax.experimental.pallas import tpu as pltpu
from jax.experimental.pallas import tpu_sc as plsc

_LANES = 16
_NBUF = 4
_AHEAD = 2


def _make_emb_kernel(B, T, D, n_workers, ch):
    rw = T // n_workers
    tok_per_w = B * rw
    n_chunks = tok_per_w // ch
    chunks_per_b = rw // ch
    mesh = plsc.VectorSubcoreMesh(core_axis_name="c", subcore_axis_name="s")
    n_cores = mesh.num_cores

    @functools.partial(
        pl.kernel,
        mesh=mesh,
        out_type=jax.ShapeDtypeStruct((B * T, D), jnp.float32),
        scratch_types=[
            pltpu.VMEM((tok_per_w,), jnp.int32),
            [pltpu.VMEM((ch, D), jnp.float32) for _ in range(_NBUF)],
            pltpu.VMEM((rw, D), jnp.float32),
            [pltpu.SemaphoreType.DMA for _ in range(_NBUF)],
            [pltpu.SemaphoreType.DMA for _ in range(_NBUF)],
            pltpu.SemaphoreType.DMA,
        ],
    )
    def emb_kernel(idx_hbm, tok_hbm, pos_hbm, out_hbm, idx_v, bufs, pos_v,
                   g_sems, o_sems, p_sem):
        wid = lax.axis_index("s") * n_cores + lax.axis_index("c")
        t0 = wid * rw
        base = wid * tok_per_w

        pltpu.sync_copy(idx_hbm.at[pl.ds(base, tok_per_w)], idx_v)
        p = pltpu.async_copy(pos_hbm.at[pl.ds(t0, rw)], pos_v, p_sem)

        def start_gather(c):
            return pltpu.async_copy(
                tok_hbm.at[idx_v.at[pl.ds(c * ch, ch)]], bufs[c % _NBUF],
                g_sems[c % _NBUF])

        def out_row(c):
            return (c // chunks_per_b) * T + t0 + (c % chunks_per_b) * ch

        gathers = {c: start_gather(c) for c in range(_AHEAD)}
        outs = {}
        p.wait()
        for c in range(n_chunks):
            nxt = c + _AHEAD
            if nxt < n_chunks:
                if nxt - _NBUF >= 0:
                    outs[nxt - _NBUF].wait()
                gathers[nxt] = start_gather(nxt)
            gathers[c].wait()
            buf = bufs[c % _NBUF]
            prow = (c % chunks_per_b) * ch

            def add_body(j, carry, _buf=buf, _prow=prow):
                col = j * _LANES
                for i in range(ch):
                    plsc.addupdate(_buf.at[i, pl.ds(col, _LANES)],
                                   pos_v[_prow + i, pl.ds(col, _LANES)])
                return carry

            lax.fori_loop(0, D // _LANES, add_body, 0)
            outs[c] = pltpu.async_copy(
                buf, out_hbm.at[pl.ds(out_row(c), ch)], o_sems[c % _NBUF])
        for c in range(max(0, n_chunks - _NBUF), n_chunks):
            outs[c].wait()

    return emb_kernel


@jax.jit
def kernel(idx, tok_emb, pos_emb):
    b, t = idx.shape
    _, d = tok_emb.shape
    n_workers = 32
    rw = t // n_workers
    idx_perm = (idx.astype(jnp.int32).reshape(b, n_workers, rw)
                .transpose(1, 0, 2).reshape(b * t))
    pos2d = pos_emb[0, :t, :].reshape(t, d)
    emb = _make_emb_kernel(b, t, d, n_workers, 16)
    out = emb(idx_perm, tok_emb, pos2d)
    return out.reshape(b, t, d)


# speedup vs baseline: 1.7422x; 1.2850x over previous
import functools

import jax
import jax.numpy as jnp
from jax import lax
from jax.experimental import pallas as pl
from jax.experimental.pallas import tpu as pltpu
from jax.experimental.pallas import tpu_sc as plsc

_LANES = 16
_NBUF = 4
_AHEAD = 2


def _make_emb_kernel(B, T, D, n_workers, ch):
    rw = T // n_workers
    tok_per_w = B * rw
    n_chunks = tok_per_w // ch
    chunks_per_b = rw // ch
    mesh = plsc.VectorSubcoreMesh(core_axis_name="c", subcore_axis_name="s")
    n_cores = mesh.num_cores

    @functools.partial(
        pl.kernel,
        mesh=mesh,
        out_type=jax.ShapeDtypeStruct((B * T, D), jnp.float32),
        scratch_types=[
            pltpu.VMEM((tok_per_w,), jnp.int32),
            [pltpu.VMEM((ch, D), jnp.float32) for _ in range(_NBUF)],
            pltpu.VMEM((rw, D), jnp.float32),
            [pltpu.SemaphoreType.DMA for _ in range(_NBUF)],
            [pltpu.SemaphoreType.DMA for _ in range(_NBUF)],
            pltpu.SemaphoreType.DMA,
        ],
    )
    def emb_kernel(idx_hbm, tok_hbm, pos_hbm, out_hbm, idx_v, bufs, pos_v,
                   g_sems, o_sems, p_sem):
        wid = lax.axis_index("s") * n_cores + lax.axis_index("c")
        t0 = wid * rw
        base = wid * tok_per_w

        pltpu.sync_copy(idx_hbm.at[pl.ds(base, tok_per_w)], idx_v)
        p = pltpu.async_copy(pos_hbm.at[pl.ds(t0, rw)], pos_v, p_sem)

        def start_gather(c):
            return pltpu.async_copy(
                tok_hbm.at[idx_v.at[pl.ds(c * ch, ch)]], bufs[c % _NBUF],
                g_sems[c % _NBUF])

        def out_row(c):
            return (c // chunks_per_b) * T + t0 + (c % chunks_per_b) * ch

        gathers = {c: start_gather(c) for c in range(_AHEAD)}
        outs = {}
        p.wait()
        for c in range(n_chunks):
            nxt = c + _AHEAD
            if nxt < n_chunks:
                if nxt - _NBUF >= 0:
                    outs[nxt - _NBUF].wait()
                gathers[nxt] = start_gather(nxt)
            gathers[c].wait()
            buf = bufs[c % _NBUF]
            prow = (c % chunks_per_b) * ch

            def _add_rows(i, carry, _buf=buf, _prow=prow):
                for j in range(D // _LANES):
                    col = j * _LANES
                    plsc.addupdate(_buf.at[i, pl.ds(col, _LANES)],
                                   pos_v[_prow + i, pl.ds(col, _LANES)])
                return carry

            lax.fori_loop(0, ch, _add_rows, 0)

            outs[c] = pltpu.async_copy(
                buf, out_hbm.at[pl.ds(out_row(c), ch)], o_sems[c % _NBUF])
        for c in range(max(0, n_chunks - _NBUF), n_chunks):
            outs[c].wait()

    return emb_kernel


@jax.jit
def kernel(idx, tok_emb, pos_emb):
    b, t = idx.shape
    _, d = tok_emb.shape
    n_workers = 32
    rw = t // n_workers
    idx_perm = (idx.astype(jnp.int32).reshape(b, n_workers, rw)
                .transpose(1, 0, 2).reshape(b * t))
    pos2d = pos_emb[0, :t, :].reshape(t, d)
    emb = _make_emb_kernel(b, t, d, n_workers, 16)
    out = emb(idx_perm, tok_emb, pos2d)
    return out.reshape(b, t, d)
